# baseline (speedup 1.0000x reference)
import jax
import jax.numpy as jnp
from jax.experimental import pallas as pl
from jax.experimental.pallas import tpu as pltpu

_NUM_CLASSES = 80
_ATTRS = 5 + _NUM_CLASSES
_INPUT_SIZE = 608.0


def _decode_body(W, H, C_ROWS):
    invW = 1.0 / W
    invH = 1.0 / H
    stride_w = _INPUT_SIZE / W
    stride_h = _INPUT_SIZE / H

    def body(x_ref, anch_ref, o_ref):
        t = pl.program_id(0)
        v = x_ref[...]
        sig = jax.nn.sigmoid(v)
        ex = jnp.exp(v)
        k = jax.lax.broadcasted_iota(jnp.int32, v.shape, 3)
        c = k % _ATTRS
        gx = jax.lax.broadcasted_iota(jnp.int32, v.shape, 1).astype(jnp.float32)
        gy = (jax.lax.broadcasted_iota(jnp.int32, v.shape, 0)
              + t * C_ROWS).astype(jnp.float32)
        a0w = anch_ref[0, 6] * (invW / stride_w)
        a1w = anch_ref[0, 7] * (invW / stride_w)
        a2w = anch_ref[0, 8] * (invW / stride_w)
        a0h = anch_ref[1, 6] * (invH / stride_h)
        a1h = anch_ref[1, 7] * (invH / stride_h)
        a2h = anch_ref[1, 8] * (invH / stride_h)
        aw = jnp.where(k < _ATTRS, a0w, jnp.where(k < 2 * _ATTRS, a1w, a2w))
        ah = jnp.where(k < _ATTRS, a0h, jnp.where(k < 2 * _ATTRS, a1h, a2h))
        o_ref[...] = jnp.where(
            c == 0, (sig + gx) * invW,
            jnp.where(
                c == 1, (sig + gy) * invH,
                jnp.where(c == 2, ex * aw,
                          jnp.where(c == 3, ex * ah, sig))))

    return body


def kernel(inputs_1, anchors):
    B, C, H, W = inputs_1.shape
    n_anch = 3
    HW = H * W
    C_ROWS = 4
    n_t = H // C_ROWS

    xt = jnp.transpose(inputs_1, (2, 3, 0, 1))
    anch_t = anchors.T

    dec = pl.pallas_call(
        _decode_body(W, H, C_ROWS),
        grid=(n_t,),
        in_specs=[
            pl.BlockSpec((C_ROWS, W, B, C), lambda t: (t, 0, 0, 0)),
            pl.BlockSpec(memory_space=pltpu.SMEM),
        ],
        out_specs=pl.BlockSpec((C_ROWS, W, B, C), lambda t: (t, 0, 0, 0)),
        out_shape=jax.ShapeDtypeStruct((H, W, B, C), jnp.float32),
    )(xt, anch_t)
    return (dec.reshape(HW, B, n_anch, _ATTRS)
               .transpose(1, 2, 0, 3)
               .reshape(B, n_anch * HW, _ATTRS))


# speedup vs baseline: 6.3715x; 6.3715x over previous
import jax
import jax.numpy as jnp
from jax.experimental import pallas as pl
from jax.experimental.pallas import tpu as pltpu

_NUM_CLASSES = 80
_ATTRS = 5 + _NUM_CLASSES
_INPUT_SIZE = 608.0


def _decode_body(W, H, C_ROWS, B, n_ch, n_anch):
    invW = 1.0 / W
    invH = 1.0 / H
    stride_w = _INPUT_SIZE / W
    stride_h = _INPUT_SIZE / H
    S = C_ROWS * W

    def body(x_ref, anch_ref, o_ref):
        t = pl.program_id(0)
        v = x_ref[...]
        r = v.reshape(S, B, n_ch)
        sig = jax.nn.sigmoid(r)
        ex = jnp.exp(r)
        k = jax.lax.broadcasted_iota(jnp.int32, r.shape, 2)
        c = k % _ATTRS
        s = jax.lax.broadcasted_iota(jnp.int32, r.shape, 0)
        gx = (s % W).astype(jnp.float32)
        gy = (t * C_ROWS + s // W).astype(jnp.float32)
        a0w = anch_ref[0, 6] * (invW / stride_w)
        a1w = anch_ref[0, 7] * (invW / stride_w)
        a2w = anch_ref[0, 8] * (invW / stride_w)
        a0h = anch_ref[1, 6] * (invH / stride_h)
        a1h = anch_ref[1, 7] * (invH / stride_h)
        a2h = anch_ref[1, 8] * (invH / stride_h)
        aw = jnp.where(k < _ATTRS, a0w, jnp.where(k < 2 * _ATTRS, a1w, a2w))
        ah = jnp.where(k < _ATTRS, a0h, jnp.where(k < 2 * _ATTRS, a1h, a2h))
        dec = jnp.where(
            c == 0, (sig + gx) * invW,
            jnp.where(
                c == 1, (sig + gy) * invH,
                jnp.where(c == 2, ex * aw,
                          jnp.where(c == 3, ex * ah, sig))))
        for a in range(n_anch):
            o_ref[:, a] = jnp.transpose(
                dec[:, :, a * _ATTRS:(a + 1) * _ATTRS], (1, 0, 2))

    return body


def kernel(inputs_1, anchors):
    B, C, H, W = inputs_1.shape
    n_anch = 3
    HW = H * W
    C_ROWS = 4
    n_t = H // C_ROWS
    S = C_ROWS * W

    xt = jnp.transpose(inputs_1, (2, 3, 0, 1))
    anch_t = anchors.T

    out = pl.pallas_call(
        _decode_body(W, H, C_ROWS, B, C, n_anch),
        grid=(n_t,),
        in_specs=[
            pl.BlockSpec((C_ROWS, W, B, C), lambda t: (t, 0, 0, 0)),
            pl.BlockSpec(memory_space=pltpu.SMEM),
        ],
        out_specs=pl.BlockSpec((B, n_anch, S, _ATTRS), lambda t: (0, 0, t, 0)),
        out_shape=jax.ShapeDtypeStruct((B, n_anch, HW, _ATTRS), jnp.float32),
    )(xt, anch_t)
    return out.reshape(B, n_anch * HW, _ATTRS)


# speedup vs baseline: 6.5685x; 1.0309x over previous
import jax
import jax.numpy as jnp
from jax.experimental import pallas as pl
from jax.experimental.pallas import tpu as pltpu

_NUM_CLASSES = 80
_ATTRS = 5 + _NUM_CLASSES
_INPUT_SIZE = 608.0


def _decode_body(W, H, C_ROWS, B, n_ch, n_anch):
    invW = 1.0 / W
    invH = 1.0 / H
    stride_w = _INPUT_SIZE / W
    stride_h = _INPUT_SIZE / H
    S = C_ROWS * W

    def body(x_ref, anch_ref, o_ref):
        t = pl.program_id(0)
        v = x_ref[...]
        r = v.reshape(S, B, n_ch)
        ex = jnp.exp(r)
        sig = ex / (1.0 + ex)
        k = jax.lax.broadcasted_iota(jnp.int32, r.shape, 2)
        c = k % _ATTRS
        s = jax.lax.broadcasted_iota(jnp.int32, r.shape, 0)
        gx = (s % W).astype(jnp.float32)
        gy = (t * C_ROWS + s // W).astype(jnp.float32)
        a0w = anch_ref[0, 6] * (invW / stride_w)
        a1w = anch_ref[0, 7] * (invW / stride_w)
        a2w = anch_ref[0, 8] * (invW / stride_w)
        a0h = anch_ref[1, 6] * (invH / stride_h)
        a1h = anch_ref[1, 7] * (invH / stride_h)
        a2h = anch_ref[1, 8] * (invH / stride_h)
        aw = jnp.where(k < _ATTRS, a0w, jnp.where(k < 2 * _ATTRS, a1w, a2w))
        ah = jnp.where(k < _ATTRS, a0h, jnp.where(k < 2 * _ATTRS, a1h, a2h))
        dec = jnp.where(
            c == 0, (sig + gx) * invW,
            jnp.where(
                c == 1, (sig + gy) * invH,
                jnp.where(c == 2, ex * aw,
                          jnp.where(c == 3, ex * ah, sig))))
        for a in range(n_anch):
            o_ref[:, a] = jnp.transpose(
                dec[:, :, a * _ATTRS:(a + 1) * _ATTRS], (1, 0, 2))

    return body


def kernel(inputs_1, anchors):
    B, C, H, W = inputs_1.shape
    n_anch = 3
    HW = H * W
    C_ROWS = 4
    n_t = H // C_ROWS
    S = C_ROWS * W

    xt = jnp.transpose(inputs_1, (2, 3, 0, 1))
    anch_t = anchors.T

    out = pl.pallas_call(
        _decode_body(W, H, C_ROWS, B, C, n_anch),
        grid=(n_t,),
        in_specs=[
            pl.BlockSpec((C_ROWS, W, B, C), lambda t: (t, 0, 0, 0)),
            pl.BlockSpec(memory_space=pltpu.SMEM),
        ],
        out_specs=pl.BlockSpec((B, n_anch, S, _ATTRS), lambda t: (0, 0, t, 0)),
        out_shape=jax.ShapeDtypeStruct((B, n_anch, HW, _ATTRS), jnp.float32),
    )(xt, anch_t)
    return out.reshape(B, n_anch * HW, _ATTRS)
